# Initial kernel scaffold; baseline (speedup 1.0000x reference)
import jax
import jax.numpy as jnp
from jax.experimental import pallas as pl


def kernel(lab, ref_l, ref_a, ref_b):
    raise NotImplementedError("write your pallas kernel here")


# speedup vs baseline: 1.7649x; 1.7649x over previous
import functools

import jax
import jax.numpy as jnp
from jax import lax
from jax.experimental import pallas as pl
from jax.experimental.pallas import tpu as pltpu
from jax.experimental.pallas import tpu_sc as plsc

NBINS = 100
B, C, H, W = 32, 3, 512, 512
NC, NS = 2, 16
LANES = 16
CHUNK_ROWS = 64
NCHUNK = H // CHUNK_ROWS
SLICES = CHUNK_ROWS * W // LANES

_LO = (0.0, -128.0, -128.0)
_BS = (1.0, 2.55, 2.55)
_HIST_W = 128
_HSZ = _HIST_W * LANES


def _top2_scan(hist_ref, base):
    def body(b, carry):
        m, am, m2, am2 = carry
        row = hist_ref[pl.ds(base + b * LANES, LANES)]
        t = jnp.sum(row)
        gt = t > m
        gt2 = jnp.logical_and(jnp.logical_not(gt), t > m2)
        m2n = jnp.where(gt, m, jnp.where(gt2, t, m2))
        am2n = jnp.where(gt, am, jnp.where(gt2, b, am2))
        mn = jnp.where(gt, t, m)
        amn = jnp.where(gt, b, am)
        return mn, amn, m2n, am2n

    init = (jnp.int32(-1), jnp.int32(0), jnp.int32(-1), jnp.int32(0))
    _, am, _, am2 = lax.fori_loop(0, NBINS, body, init)
    return am, am2


def _peak_value(bin_idx, lo, bs):
    bc = lax.broadcast_in_dim(bin_idx, (LANES,), ())
    f = bc.astype(jnp.float32)
    return lo + (f + 0.5) * bs


def _sc_body(lab_hbm, peaks_hbm, buf, hist, stage):
    w = lax.axis_index("s") * NC + lax.axis_index("c")
    liota = lax.iota(jnp.int32, LANES)
    ones = jnp.ones((LANES,), jnp.int32)
    zeros = jnp.zeros((LANES,), jnp.int32)

    def zbody(i, _):
        hist[pl.ds(i * LANES, LANES)] = zeros
        return 0
    lax.fori_loop(0, C * _HIST_W, zbody, 0)

    for c in range(C):
        lo = jnp.float32(_LO[c])
        bs = jnp.float32(_BS[c])
        base = c * _HSZ

        def chunk_body(k, _, c=c, lo=lo, bs=bs, base=base):
            pltpu.sync_copy(lab_hbm.at[w, c, pl.ds(k * CHUNK_ROWS, CHUNK_ROWS)],
                            buf)

            def slice_body(i, _):
                r = i >> 5
                col = (i & 31) * LANES
                v = buf[r, pl.ds(col, LANES)]
                t = (v - lo) / bs
                ti = t.astype(jnp.int32)
                ti = jnp.minimum(jnp.maximum(ti, 0), NBINS - 1)
                addr = base + ti * LANES + liota
                plsc.addupdate_scatter(hist, [addr], ones)
                return 0

            lax.fori_loop(0, SLICES, slice_body, 0)
            return 0

        lax.fori_loop(0, NCHUNK, chunk_body, 0)

    _, amL2 = _top2_scan(hist, 0 * _HSZ)
    amA, _ = _top2_scan(hist, 1 * _HSZ)
    amB, _ = _top2_scan(hist, 2 * _HSZ)
    pkL = _peak_value(amL2, _LO[0], _BS[0])
    pkA = _peak_value(amA, _LO[1], _BS[1])
    pkB = _peak_value(amB, _LO[2], _BS[2])
    out = jnp.where(liota == 0, pkL,
                    jnp.where(liota == 1, pkA,
                              jnp.where(liota == 2, pkB,
                                        jnp.float32(0.0))))
    stage[...] = out
    pltpu.sync_copy(stage, peaks_hbm.at[w])


@jax.jit
def _sc_peaks(lab):
    mesh = plsc.VectorSubcoreMesh(core_axis_name="c", subcore_axis_name="s",
                                  num_cores=NC, num_subcores=NS)
    return pl.kernel(
        _sc_body,
        out_type=jax.ShapeDtypeStruct((B, LANES), jnp.float32),
        mesh=mesh,
        scratch_types=[
            pltpu.VMEM((CHUNK_ROWS, W), jnp.float32),
            pltpu.VMEM((C * _HSZ,), jnp.int32),
            pltpu.VMEM((LANES,), jnp.float32),
        ],
        compiler_params=pltpu.CompilerParams(needs_layout_passes=False),
    )(lab)


def _norm_body(s_ref, o_ref, x_ref, out_ref):
    b = pl.program_id(0)
    c = pl.program_id(1)
    out_ref[...] = x_ref[...] * s_ref[b, c] + o_ref[b, c]


@jax.jit
def _tc_norm(scale, off, lab):
    return pl.pallas_call(
        _norm_body,
        grid=(B, C),
        in_specs=[
            pl.BlockSpec(memory_space=pltpu.SMEM),
            pl.BlockSpec(memory_space=pltpu.SMEM),
            pl.BlockSpec((1, 1, H, W), lambda b, c: (b, c, 0, 0)),
        ],
        out_specs=pl.BlockSpec((1, 1, H, W), lambda b, c: (b, c, 0, 0)),
        out_shape=jax.ShapeDtypeStruct((B, C, H, W), jnp.float32),
    )(scale, off, lab)


def kernel(lab, ref_l, ref_a, ref_b):
    peaks16 = _sc_peaks(lab)
    pk = peaks16[:, :C]
    refs = jnp.concatenate([ref_l, ref_a, ref_b]).astype(jnp.float32)
    den = jnp.array([200.0, 255.0, 255.0], jnp.float32)
    add = jnp.array([100.0, 128.0, 128.0], jnp.float32)
    scale = refs[None, :] / (pk * den[None, :])
    off = jnp.broadcast_to((add / den)[None, :], (B, C)).astype(jnp.float32)
    return _tc_norm(scale, off, lab)


# speedup vs baseline: 2.5852x; 1.4648x over previous
import numpy as np

import jax
import jax.numpy as jnp
from jax import lax
from jax.experimental import pallas as pl
from jax.experimental.pallas import tpu as pltpu
from jax.experimental.pallas import tpu_sc as plsc

NBINS = 100
B, C, H, W = 32, 3, 512, 512
NPIX = H * W
NC, NS = 2, 16
LANES = 16
CHUNK = 32768
NCHUNK = NPIX // CHUNK
UNROLL = 8
GROUPS = CHUNK // (LANES * UNROLL)

_LO = (0.0, -128.0, -128.0)
_INV = (1.0,
        float(np.float32(1.0) / np.float32(2.55)),
        float(np.float32(1.0) / np.float32(2.55)))
_BS = (1.0, 2.55, 2.55)
_HIST_W = 128
_HSZ = _HIST_W * LANES


def _top2_scan(hist_ref, base):
    def body(b, carry):
        m, am, m2, am2 = carry
        row = hist_ref[pl.ds(base + b * LANES, LANES)]
        t = jnp.sum(row)
        gt = t > m
        gt2 = jnp.logical_and(jnp.logical_not(gt), t > m2)
        m2n = jnp.where(gt, m, jnp.where(gt2, t, m2))
        am2n = jnp.where(gt, am, jnp.where(gt2, b, am2))
        mn = jnp.where(gt, t, m)
        amn = jnp.where(gt, b, am)
        return mn, amn, m2n, am2n

    init = (jnp.int32(-1), jnp.int32(0), jnp.int32(-1), jnp.int32(0))
    _, am, _, am2 = lax.fori_loop(0, NBINS, body, init)
    return am, am2


def _peak_value(bin_idx, lo, bs):
    bc = lax.broadcast_in_dim(bin_idx, (LANES,), ())
    f = bc.astype(jnp.float32)
    return lo + (f + 0.5) * bs


def _sc_body(lab_hbm, peaks_hbm, buf0, buf1, hist, stage, sem0, sem1):
    w = lax.axis_index("s") * NC + lax.axis_index("c")
    liota = lax.iota(jnp.int32, LANES)
    ones = jnp.ones((LANES,), jnp.int32)
    zeros = jnp.zeros((LANES,), jnp.int32)
    bufs = (buf0, buf1)
    sems = (sem0, sem1)

    def zbody(i, _):
        hist[pl.ds(i * LANES, LANES)] = zeros
        return 0
    lax.fori_loop(0, C * _HIST_W, zbody, 0)

    def compute_chunk(bref, c, lo, inv, base):
        @plsc.parallel_loop(0, CHUNK // LANES, 1, unroll=UNROLL)
        def _(i):
            v = bref[pl.ds(i * LANES, LANES)]
            if c == 0:
                t = v
            else:
                t = (v - lo) * inv
            ti = t.astype(jnp.int32)
            ti = jnp.minimum(ti, NBINS - 1)
            addr = base + ti * LANES + liota
            plsc.addupdate_scatter(hist, [addr], ones)

    for c in range(C):
        lo = jnp.float32(_LO[c])
        inv = jnp.float32(_INV[c])
        base = c * _HSZ

        def copy(k):
            return pltpu.make_async_copy(
                lab_hbm.at[w * C + c, 0, pl.ds(k * CHUNK, CHUNK)],
                bufs[k % 2], sems[k % 2])

        copy(0).start()
        for k in range(NCHUNK):
            if k + 1 < NCHUNK:
                copy(k + 1).start()
            copy(k).wait()
            compute_chunk(bufs[k % 2], c, lo, inv, base)

    _, amL2 = _top2_scan(hist, 0 * _HSZ)
    amA, _ = _top2_scan(hist, 1 * _HSZ)
    amB, _ = _top2_scan(hist, 2 * _HSZ)
    pkL = _peak_value(amL2, _LO[0], _BS[0])
    pkA = _peak_value(amA, _LO[1], _BS[1])
    pkB = _peak_value(amB, _LO[2], _BS[2])
    out = jnp.where(liota == 0, pkL,
                    jnp.where(liota == 1, pkA,
                              jnp.where(liota == 2, pkB,
                                        jnp.float32(0.0))))
    stage[...] = out
    pltpu.sync_copy(stage, peaks_hbm.at[w])


@jax.jit
def _sc_peaks(lab3):
    mesh = plsc.VectorSubcoreMesh(core_axis_name="c", subcore_axis_name="s",
                                  num_cores=NC, num_subcores=NS)
    return pl.kernel(
        _sc_body,
        out_type=jax.ShapeDtypeStruct((B, LANES), jnp.float32),
        mesh=mesh,
        scratch_types=[
            pltpu.VMEM((CHUNK,), jnp.float32),
            pltpu.VMEM((CHUNK,), jnp.float32),
            pltpu.VMEM((C * _HSZ,), jnp.int32),
            pltpu.VMEM((LANES,), jnp.float32),
            pltpu.SemaphoreType.DMA,
            pltpu.SemaphoreType.DMA,
        ],
        compiler_params=pltpu.CompilerParams(needs_layout_passes=False),
    )(lab3)


def _norm_body(s_ref, o_ref, x_ref, out_ref):
    b = pl.program_id(0)
    c = pl.program_id(1)
    out_ref[...] = x_ref[...] * s_ref[b, c] + o_ref[b, c]


@jax.jit
def _tc_norm(scale, off, lab):
    return pl.pallas_call(
        _norm_body,
        grid=(B, C),
        in_specs=[
            pl.BlockSpec(memory_space=pltpu.SMEM),
            pl.BlockSpec(memory_space=pltpu.SMEM),
            pl.BlockSpec((1, 1, H, W), lambda b, c: (b, c, 0, 0)),
        ],
        out_specs=pl.BlockSpec((1, 1, H, W), lambda b, c: (b, c, 0, 0)),
        out_shape=jax.ShapeDtypeStruct((B, C, H, W), jnp.float32),
    )(scale, off, lab)


def kernel(lab, ref_l, ref_a, ref_b):
    peaks16 = _sc_peaks(lab.reshape(B * C, 1, NPIX))
    pk = peaks16[:, :C]
    refs = jnp.concatenate([ref_l, ref_a, ref_b]).astype(jnp.float32)
    den = jnp.array([200.0, 255.0, 255.0], jnp.float32)
    add = jnp.array([100.0, 128.0, 128.0], jnp.float32)
    scale = refs[None, :] / (pk * den[None, :])
    off = jnp.broadcast_to((add / den)[None, :], (B, C)).astype(jnp.float32)
    return _tc_norm(scale, off, lab)


# speedup vs baseline: 6.4740x; 2.5043x over previous
import numpy as np

import jax
import jax.numpy as jnp
from jax import lax
from jax.experimental import pallas as pl
from jax.experimental.pallas import tpu as pltpu
from jax.experimental.pallas import tpu_sc as plsc

NBINS = 100
B, C, H, W = 32, 3, 512, 512
NPIX = H * W
NC, NS = 2, 16
LANES = 16
CROWS = 64
CHUNK = CROWS * W
NCHUNK = H // CROWS
UNROLL = 8
GROUPS = CHUNK // (LANES * UNROLL)

_LO = (0.0, -128.0, -128.0)
_INV = (1.0,
        float(np.float32(1.0) / np.float32(2.55)),
        float(np.float32(1.0) / np.float32(2.55)))
_BS = (1.0, 2.55, 2.55)
_HIST_W = 128
_HSZ = _HIST_W * LANES


def _top2_scan(hist_ref, base):
    def body(b, carry):
        m, am, m2, am2 = carry
        row = hist_ref[pl.ds(base + b * LANES, LANES)]
        t = jnp.sum(row)
        gt = t > m
        gt2 = jnp.logical_and(jnp.logical_not(gt), t > m2)
        m2n = jnp.where(gt, m, jnp.where(gt2, t, m2))
        am2n = jnp.where(gt, am, jnp.where(gt2, b, am2))
        mn = jnp.where(gt, t, m)
        amn = jnp.where(gt, b, am)
        return mn, amn, m2n, am2n

    init = (jnp.int32(-1), jnp.int32(0), jnp.int32(-1), jnp.int32(0))
    _, am, _, am2 = lax.fori_loop(0, NBINS, body, init)
    return am, am2


def _peak_value(bin_idx, lo, bs):
    bc = lax.broadcast_in_dim(bin_idx, (LANES,), ())
    f = bc.astype(jnp.float32)
    return lo + (f + 0.5) * bs


def _sc_body(lab_hbm, peaks_hbm, buf0, buf1, hist, stage, sem0, sem1):
    w = lax.axis_index("s") * NC + lax.axis_index("c")
    liota = lax.iota(jnp.int32, LANES)
    ones = jnp.ones((LANES,), jnp.int32)
    zeros = jnp.zeros((LANES,), jnp.int32)
    bufs = (buf0, buf1)
    sems = (sem0, sem1)

    def zbody(i, _):
        hist[pl.ds(i * LANES, LANES)] = zeros
        return 0
    lax.fori_loop(0, C * _HIST_W, zbody, 0)

    def compute_chunk(bref, c, lo, inv, base):
        @plsc.parallel_loop(0, CHUNK // LANES, 1, unroll=UNROLL)
        def _(i):
            r = i >> 5
            col = (i & 31) * LANES
            v = bref[r, pl.ds(col, LANES)]
            if c == 0:
                t = v
            else:
                t = (v - lo) * inv
            ti = t.astype(jnp.int32)
            ti = jnp.minimum(ti, NBINS - 1)
            addr = base + ti * LANES + liota
            plsc.addupdate_scatter(hist, [addr], ones)

    for c in range(C):
        lo = jnp.float32(_LO[c])
        inv = jnp.float32(_INV[c])
        base = c * _HSZ

        def copy(k):
            return pltpu.make_async_copy(
                lab_hbm.at[w, c, pl.ds(k * CROWS, CROWS)],
                bufs[k % 2], sems[k % 2])

        copy(0).start()
        for k in range(NCHUNK):
            if k + 1 < NCHUNK:
                copy(k + 1).start()
            copy(k).wait()
            compute_chunk(bufs[k % 2], c, lo, inv, base)

    _, amL2 = _top2_scan(hist, 0 * _HSZ)
    amA, _ = _top2_scan(hist, 1 * _HSZ)
    amB, _ = _top2_scan(hist, 2 * _HSZ)
    pkL = _peak_value(amL2, _LO[0], _BS[0])
    pkA = _peak_value(amA, _LO[1], _BS[1])
    pkB = _peak_value(amB, _LO[2], _BS[2])
    out = jnp.where(liota == 0, pkL,
                    jnp.where(liota == 1, pkA,
                              jnp.where(liota == 2, pkB,
                                        jnp.float32(0.0))))
    stage[...] = out
    pltpu.sync_copy(stage, peaks_hbm.at[w])


@jax.jit
def _sc_peaks(lab3):
    mesh = plsc.VectorSubcoreMesh(core_axis_name="c", subcore_axis_name="s",
                                  num_cores=NC, num_subcores=NS)
    return pl.kernel(
        _sc_body,
        out_type=jax.ShapeDtypeStruct((B, LANES), jnp.float32),
        mesh=mesh,
        scratch_types=[
            pltpu.VMEM((CROWS, W), jnp.float32),
            pltpu.VMEM((CROWS, W), jnp.float32),
            pltpu.VMEM((C * _HSZ,), jnp.int32),
            pltpu.VMEM((LANES,), jnp.float32),
            pltpu.SemaphoreType.DMA,
            pltpu.SemaphoreType.DMA,
        ],
        compiler_params=pltpu.CompilerParams(needs_layout_passes=False),
    )(lab3)


def _norm_body(s_ref, o_ref, x_ref, out_ref):
    b = pl.program_id(0)
    c = pl.program_id(1)
    out_ref[...] = x_ref[...] * s_ref[b, c] + o_ref[b, c]


@jax.jit
def _tc_norm(scale, off, lab):
    return pl.pallas_call(
        _norm_body,
        grid=(B, C),
        in_specs=[
            pl.BlockSpec(memory_space=pltpu.SMEM),
            pl.BlockSpec(memory_space=pltpu.SMEM),
            pl.BlockSpec((1, 1, H, W), lambda b, c: (b, c, 0, 0)),
        ],
        out_specs=pl.BlockSpec((1, 1, H, W), lambda b, c: (b, c, 0, 0)),
        out_shape=jax.ShapeDtypeStruct((B, C, H, W), jnp.float32),
    )(scale, off, lab)


def kernel(lab, ref_l, ref_a, ref_b):
    peaks16 = _sc_peaks(lab)
    pk = peaks16[:, :C]
    refs = jnp.concatenate([ref_l, ref_a, ref_b]).astype(jnp.float32)
    den = jnp.array([200.0, 255.0, 255.0], jnp.float32)
    add = jnp.array([100.0, 128.0, 128.0], jnp.float32)
    scale = refs[None, :] / (pk * den[None, :])
    off = jnp.broadcast_to((add / den)[None, :], (B, C)).astype(jnp.float32)
    return _tc_norm(scale, off, lab)


# speedup vs baseline: 6.4798x; 1.0009x over previous
import numpy as np

import jax
import jax.numpy as jnp
from jax import lax
from jax.experimental import pallas as pl
from jax.experimental.pallas import tpu as pltpu
from jax.experimental.pallas import tpu_sc as plsc

NBINS = 100
B, C, H, W = 32, 3, 512, 512
NPIX = H * W
NC, NS = 2, 16
LANES = 16
CROWS = 64
CHUNK = CROWS * W
NCHUNK = H // CROWS
UNROLL = 8
GROUPS = CHUNK // (LANES * UNROLL)

_LO = (0.0, -128.0, -128.0)
_INV = (1.0,
        float(np.float32(1.0) / np.float32(2.55)),
        float(np.float32(1.0) / np.float32(2.55)))
_BS = (1.0, 2.55, 2.55)
_HIST_W = 128
_HSZ = _HIST_W * LANES


def _top2_scan(hist_ref, base):
    def body(b, carry):
        m, am, m2, am2 = carry
        row = hist_ref[pl.ds(base + b * LANES, LANES)]
        t = jnp.sum(row)
        gt = t > m
        gt2 = jnp.logical_and(jnp.logical_not(gt), t > m2)
        m2n = jnp.where(gt, m, jnp.where(gt2, t, m2))
        am2n = jnp.where(gt, am, jnp.where(gt2, b, am2))
        mn = jnp.where(gt, t, m)
        amn = jnp.where(gt, b, am)
        return mn, amn, m2n, am2n

    init = (jnp.int32(-1), jnp.int32(0), jnp.int32(-1), jnp.int32(0))
    _, am, _, am2 = lax.fori_loop(0, NBINS, body, init)
    return am, am2


def _peak_value(bin_idx, lo, bs):
    bc = lax.broadcast_in_dim(bin_idx, (LANES,), ())
    f = bc.astype(jnp.float32)
    return lo + (f + 0.5) * bs


def _sc_body(lab_hbm, peaks_hbm, buf0, buf1, hist, stage, sem0, sem1):
    w = lax.axis_index("s") * NC + lax.axis_index("c")
    liota = lax.iota(jnp.int32, LANES)
    ones = jnp.ones((LANES,), jnp.int32)
    zeros = jnp.zeros((LANES,), jnp.int32)
    bufs = (buf0, buf1)
    sems = (sem0, sem1)

    def zbody(i, _):
        hist[pl.ds(i * LANES, LANES)] = zeros
        return 0
    lax.fori_loop(0, C * _HIST_W, zbody, 0)

    def compute_chunk(bref, c, lo, inv, base):
        @plsc.parallel_loop(0, CHUNK // LANES, 1, unroll=UNROLL)
        def _(i):
            r = i >> 5
            col = (i & 31) * LANES
            v = bref[r, pl.ds(col, LANES)]
            if c == 0:
                t = v
            else:
                t = (v - lo) * inv
            ti = t.astype(jnp.int32)
            ti = jnp.minimum(ti, NBINS - 1)
            addr = base + ti * LANES + liota
            plsc.addupdate_scatter(hist, [addr], ones)

    for c in range(C):
        lo = jnp.float32(_LO[c])
        inv = jnp.float32(_INV[c])
        base = c * _HSZ

        def copy(k):
            return pltpu.make_async_copy(
                lab_hbm.at[w, c, pl.ds(k * CROWS, CROWS)],
                bufs[k % 2], sems[k % 2])

        copy(0).start()
        for k in range(NCHUNK):
            if k + 1 < NCHUNK:
                copy(k + 1).start()
            copy(k).wait()
            compute_chunk(bufs[k % 2], c, lo, inv, base)

    _, amL2 = _top2_scan(hist, 0 * _HSZ)
    amA, _ = _top2_scan(hist, 1 * _HSZ)
    amB, _ = _top2_scan(hist, 2 * _HSZ)
    pkL = _peak_value(amL2, _LO[0], _BS[0])
    pkA = _peak_value(amA, _LO[1], _BS[1])
    pkB = _peak_value(amB, _LO[2], _BS[2])
    out = jnp.where(liota == 0, pkL,
                    jnp.where(liota == 1, pkA,
                              jnp.where(liota == 2, pkB,
                                        jnp.float32(0.0))))
    stage[...] = out
    pltpu.sync_copy(stage, peaks_hbm.at[w])


@jax.jit
def _sc_peaks(lab3):
    mesh = plsc.VectorSubcoreMesh(core_axis_name="c", subcore_axis_name="s",
                                  num_cores=NC, num_subcores=NS)
    return pl.kernel(
        _sc_body,
        out_type=jax.ShapeDtypeStruct((B, LANES), jnp.float32),
        mesh=mesh,
        scratch_types=[
            pltpu.VMEM((CROWS, W), jnp.float32),
            pltpu.VMEM((CROWS, W), jnp.float32),
            pltpu.VMEM((C * _HSZ,), jnp.int32),
            pltpu.VMEM((LANES,), jnp.float32),
            pltpu.SemaphoreType.DMA,
            pltpu.SemaphoreType.DMA,
        ],
        compiler_params=pltpu.CompilerParams(needs_layout_passes=False),
    )(lab3)


def _norm_body(pk_ref, rl_ref, ra_ref, rb_ref, x_ref, out_ref):
    b = pl.program_id(0)
    c = pl.program_id(1)
    pk = pk_ref[b, c]
    refc = jnp.where(c == 0, rl_ref[0], jnp.where(c == 1, ra_ref[0],
                                                  rb_ref[0]))
    den = jnp.where(c == 0, jnp.float32(200.0), jnp.float32(255.0))
    off = jnp.where(c == 0, jnp.float32(0.5), jnp.float32(128.0 / 255.0))
    s = refc / (pk * den)
    out_ref[...] = x_ref[...] * s + off


@jax.jit
def _tc_norm(peaks16, ref_l, ref_a, ref_b, lab):
    return pl.pallas_call(
        _norm_body,
        grid=(B, C),
        in_specs=[
            pl.BlockSpec(memory_space=pltpu.SMEM),
            pl.BlockSpec(memory_space=pltpu.SMEM),
            pl.BlockSpec(memory_space=pltpu.SMEM),
            pl.BlockSpec(memory_space=pltpu.SMEM),
            pl.BlockSpec((1, 1, H, W), lambda b, c: (b, c, 0, 0)),
        ],
        out_specs=pl.BlockSpec((1, 1, H, W), lambda b, c: (b, c, 0, 0)),
        out_shape=jax.ShapeDtypeStruct((B, C, H, W), jnp.float32),
    )(peaks16, ref_l, ref_a, ref_b, lab)


def kernel(lab, ref_l, ref_a, ref_b):
    peaks16 = _sc_peaks(lab)
    return _tc_norm(peaks16, ref_l, ref_a, ref_b, lab)
